# baseline (speedup 1.0000x reference)
import jax
import jax.numpy as jnp
from jax.experimental import pallas as pl
from jax.experimental.pallas import tpu as pltpu

_BN_EPS = 1e-5


def _stats_kernel(x_ref, gram_ref, sx_ref, xbf_ref):
    nb, c, _ = x_ref.shape
    g = jnp.zeros((c, c), jnp.float32)
    sx = jnp.zeros((c, 1), jnp.float32)
    for b in range(nb):
        xb = x_ref[b]
        xbf = xb.astype(jnp.bfloat16)
        xbf_ref[b] = xbf
        g = g + jax.lax.dot_general(
            xbf, xbf, (((1,), (1,)), ((), ())),
            preferred_element_type=jnp.float32)
        sx = sx + jnp.sum(xb, axis=1, keepdims=True)
    gram_ref[0] = g
    sx_ref[0] = sx


def _apply_kernel(x_ref, w1s_ref, w2_ref, shift_ref, o_ref):
    nb, _, tl = x_ref.shape
    w1s = w1s_ref[...]
    w2 = w2_ref[...]
    shift = shift_ref[...]
    lc = 2048 if tl % 2048 == 0 else tl
    for b in range(nb):
        for j in range(tl // lc):
            sl = slice(j * lc, (j + 1) * lc)
            xc = x_ref[b, :, sl]
            h = jnp.dot(w1s, xc, preferred_element_type=jnp.float32)
            h = jnp.maximum(h + shift, 0.0).astype(jnp.bfloat16)
            o_ref[b, :, sl] = jnp.dot(w2, h, preferred_element_type=jnp.float32)


def kernel(x_ncl, w1, gamma, beta, w2):
    N, C, L = x_ncl.shape
    H = w1.shape[0]
    C_out = w2.shape[0]

    max_tl = 8192
    if L <= max_tl:
        tl, n_lt = L, 1
    else:
        n_lt = int(pl.cdiv(L, max_tl))
        tl = int(pl.cdiv(int(pl.cdiv(L, n_lt)), 128)) * 128
    L_pad = n_lt * tl

    nb_stats = 1
    for cand in (4, 2, 1):
        if N % cand == 0 and (N // cand) >= 2:
            nb_stats = cand
            break
    n_bt_s = N // nb_stats

    nb_apply = 2 if N % 2 == 0 else 1
    n_bt_a = N // nb_apply

    if L_pad != L:
        x_p = jnp.pad(x_ncl, ((0, 0), (0, 0), (0, L_pad - L)))
    else:
        x_p = x_ncl

    grid_s = (n_bt_s, n_lt)
    n_chunks = n_bt_s * n_lt
    gparts, sxparts, x_bf = pl.pallas_call(
        _stats_kernel,
        out_shape=(jax.ShapeDtypeStruct((n_chunks, C, C), jnp.float32),
                   jax.ShapeDtypeStruct((n_chunks, C, 1), jnp.float32),
                   jax.ShapeDtypeStruct((N, C, L_pad), jnp.bfloat16)),
        grid=grid_s,
        in_specs=[pl.BlockSpec((nb_stats, C, tl), lambda n, l: (n, 0, l))],
        out_specs=(pl.BlockSpec((1, C, C), lambda n, l, _g=n_lt: (n * _g + l, 0, 0)),
                   pl.BlockSpec((1, C, 1), lambda n, l, _g=n_lt: (n * _g + l, 0, 0)),
                   pl.BlockSpec((nb_stats, C, tl), lambda n, l: (n, 0, l))),
        compiler_params=pltpu.CompilerParams(
            dimension_semantics=("parallel", "parallel"),
            vmem_limit_bytes=100 * 1024 * 1024),
    )(x_p)

    m_total = float(N * L)
    gram = jnp.sum(gparts, axis=0)
    sx = jnp.sum(sxparts, axis=0)[:, 0]
    w1f = w1.astype(jnp.float32)
    hp = jax.lax.Precision.HIGHEST
    sum_h = jnp.matmul(w1f, sx, precision=hp)
    mean = sum_h / m_total
    t = jnp.matmul(w1f, gram, precision=hp)
    sumsq = jnp.sum(t * w1f, axis=1)
    var = jnp.maximum(sumsq / m_total - mean * mean, 0.0)
    scale = gamma.astype(jnp.float32) * jax.lax.rsqrt(var + _BN_EPS)
    shift = (beta.astype(jnp.float32) - mean * scale).reshape(H, 1)
    w1s = (w1f * scale[:, None]).astype(jnp.bfloat16)
    w2b = w2.astype(jnp.bfloat16)

    grid_a = (n_bt_a, n_lt)
    out_p = pl.pallas_call(
        _apply_kernel,
        out_shape=jax.ShapeDtypeStruct((N, C_out, L_pad), x_ncl.dtype),
        grid=grid_a,
        in_specs=[
            pl.BlockSpec((nb_apply, C, tl), lambda n, l: (n, 0, l)),
            pl.BlockSpec((H, C), lambda n, l: (0, 0)),
            pl.BlockSpec((C_out, H), lambda n, l: (0, 0)),
            pl.BlockSpec((H, 1), lambda n, l: (0, 0)),
        ],
        out_specs=pl.BlockSpec((nb_apply, C_out, tl), lambda n, l: (n, 0, l)),
        compiler_params=pltpu.CompilerParams(
            dimension_semantics=("parallel", "parallel"),
            vmem_limit_bytes=100 * 1024 * 1024),
    )(x_bf, w1s, w2b, shift)

    if L_pad != L:
        out_p = out_p[:, :, :L]
    return out_p


# speedup vs baseline: 1.0800x; 1.0800x over previous
import jax
import jax.numpy as jnp
from jax.experimental import pallas as pl
from jax.experimental.pallas import tpu as pltpu

_BN_EPS = 1e-5


def _stats_kernel(x_ref, gram_ref, sx_ref):
    nb, c, _ = x_ref.shape
    g = jnp.zeros((c, c), jnp.float32)
    sx = jnp.zeros((c, 1), jnp.float32)
    for b in range(nb):
        xb = x_ref[b]
        xbf = xb.astype(jnp.bfloat16)
        g = g + jax.lax.dot_general(
            xbf, xbf, (((1,), (1,)), ((), ())),
            preferred_element_type=jnp.float32)
        sx = sx + jnp.sum(xb, axis=1, keepdims=True)
    gram_ref[0] = g
    sx_ref[0] = sx


def _apply_kernel(x_ref, w1s_ref, w2_ref, shift_ref, o_ref):
    nb, _, tl = x_ref.shape
    w1s = w1s_ref[...]
    w2 = w2_ref[...]
    shift = shift_ref[...]
    lc = 4096 if tl % 4096 == 0 else tl
    for b in range(nb):
        for j in range(tl // lc):
            sl = slice(j * lc, (j + 1) * lc)
            xc = x_ref[b, :, sl].astype(jnp.bfloat16)
            h = jnp.dot(w1s, xc, preferred_element_type=jnp.float32)
            h = jnp.maximum(h + shift, 0.0).astype(jnp.bfloat16)
            o_ref[b, :, sl] = jnp.dot(w2, h, preferred_element_type=jnp.float32)


def kernel(x_ncl, w1, gamma, beta, w2):
    N, C, L = x_ncl.shape
    H = w1.shape[0]
    C_out = w2.shape[0]

    max_tl = 8192
    if L <= max_tl:
        tl, n_lt = L, 1
    else:
        n_lt = int(pl.cdiv(L, max_tl))
        tl = int(pl.cdiv(int(pl.cdiv(L, n_lt)), 128)) * 128
    L_pad = n_lt * tl

    nb_stats = 1
    for cand in (4, 2, 1):
        if N % cand == 0 and (N // cand) >= 2:
            nb_stats = cand
            break
    n_bt_s = N // nb_stats

    nb_apply = 2 if N % 2 == 0 else 1
    n_bt_a = N // nb_apply

    if L_pad != L:
        x_p = jnp.pad(x_ncl, ((0, 0), (0, 0), (0, L_pad - L)))
    else:
        x_p = x_ncl

    grid_s = (n_bt_s, n_lt)
    n_chunks = n_bt_s * n_lt
    gparts, sxparts = pl.pallas_call(
        _stats_kernel,
        out_shape=(jax.ShapeDtypeStruct((n_chunks, C, C), jnp.float32),
                   jax.ShapeDtypeStruct((n_chunks, C, 1), jnp.float32)),
        grid=grid_s,
        in_specs=[pl.BlockSpec((nb_stats, C, tl), lambda n, l: (n, 0, l))],
        out_specs=(pl.BlockSpec((1, C, C), lambda n, l, _g=n_lt: (n * _g + l, 0, 0)),
                   pl.BlockSpec((1, C, 1), lambda n, l, _g=n_lt: (n * _g + l, 0, 0))),
        compiler_params=pltpu.CompilerParams(
            dimension_semantics=("parallel", "parallel"),
            vmem_limit_bytes=100 * 1024 * 1024),
    )(x_p)

    m_total = float(N * L)
    gram = jnp.sum(gparts, axis=0)
    sx = jnp.sum(sxparts, axis=0)[:, 0]
    w1f = w1.astype(jnp.float32)
    hp = jax.lax.Precision.HIGHEST
    sum_h = jnp.matmul(w1f, sx, precision=hp)
    mean = sum_h / m_total
    t = jnp.matmul(w1f, gram, precision=hp)
    sumsq = jnp.sum(t * w1f, axis=1)
    var = jnp.maximum(sumsq / m_total - mean * mean, 0.0)
    scale = gamma.astype(jnp.float32) * jax.lax.rsqrt(var + _BN_EPS)
    shift = (beta.astype(jnp.float32) - mean * scale).reshape(H, 1)
    w1s = (w1f * scale[:, None]).astype(jnp.bfloat16)
    w2b = w2.astype(jnp.bfloat16)

    grid_a = (n_bt_a, n_lt)
    out_p = pl.pallas_call(
        _apply_kernel,
        out_shape=jax.ShapeDtypeStruct((N, C_out, L_pad), x_ncl.dtype),
        grid=grid_a,
        in_specs=[
            pl.BlockSpec((nb_apply, C, tl), lambda n, l: (n, 0, l)),
            pl.BlockSpec((H, C), lambda n, l: (0, 0)),
            pl.BlockSpec((C_out, H), lambda n, l: (0, 0)),
            pl.BlockSpec((H, 1), lambda n, l: (0, 0)),
        ],
        out_specs=pl.BlockSpec((nb_apply, C_out, tl), lambda n, l: (n, 0, l)),
        compiler_params=pltpu.CompilerParams(
            dimension_semantics=("parallel", "parallel"),
            vmem_limit_bytes=100 * 1024 * 1024),
    )(x_p, w1s, w2b, shift)

    if L_pad != L:
        out_p = out_p[:, :, :L]
    return out_p


# speedup vs baseline: 1.1156x; 1.0329x over previous
import jax
import jax.numpy as jnp
from jax.experimental import pallas as pl
from jax.experimental.pallas import tpu as pltpu

_BN_EPS = 1e-5


def _stats_kernel(x_ref, gram_ref, sx_ref):
    nb, c, _ = x_ref.shape
    g = jnp.zeros((c, c), jnp.float32)
    sx = jnp.zeros((c, 1), jnp.float32)
    for b in range(nb):
        xb = x_ref[b]
        xbf = xb.astype(jnp.bfloat16)
        g = g + jax.lax.dot_general(
            xbf, xbf, (((1,), (1,)), ((), ())),
            preferred_element_type=jnp.float32)
        sx = sx + jnp.sum(xb, axis=1, keepdims=True)
    gram_ref[0] = g
    sx_ref[0] = sx


def _apply_kernel(x_ref, w1s_ref, w2_ref, shift_ref, o_ref):
    nb, _, tl = x_ref.shape
    w1s = w1s_ref[...]
    w2 = w2_ref[...]
    shift = shift_ref[...]
    lc = 4096 if tl % 4096 == 0 else tl
    for b in range(nb):
        for j in range(tl // lc):
            sl = slice(j * lc, (j + 1) * lc)
            xc = x_ref[b, :, sl].astype(jnp.bfloat16)
            h = jnp.dot(w1s, xc, preferred_element_type=jnp.float32)
            h = jnp.maximum(h + shift, 0.0).astype(jnp.bfloat16)
            o_ref[b, :, sl] = jnp.dot(w2, h, preferred_element_type=jnp.float32)


def _fold_kernel(inv_m, gp_ref, sxp_ref, w1_ref, w2_ref, gamma_ref, beta_ref,
                 w1s_ref, shift_ref, w2b_ref):
    gram = jnp.sum(gp_ref[...], axis=0)
    sx = jnp.sum(sxp_ref[...], axis=0)
    w1f = w1_ref[...]
    mean = jnp.dot(w1f, sx, preferred_element_type=jnp.float32) * inv_m
    t = jnp.dot(w1f, gram, preferred_element_type=jnp.float32)
    sumsq = jnp.sum(t * w1f, axis=1, keepdims=True)
    var = jnp.maximum(sumsq * inv_m - mean * mean, 0.0)
    scale = gamma_ref[...] * jax.lax.rsqrt(var + _BN_EPS)
    shift_ref[...] = beta_ref[...] - mean * scale
    w1s_ref[...] = (w1f * scale).astype(jnp.bfloat16)
    w2b_ref[...] = w2_ref[...].astype(jnp.bfloat16)


def kernel(x_ncl, w1, gamma, beta, w2):
    N, C, L = x_ncl.shape
    H = w1.shape[0]
    C_out = w2.shape[0]

    max_tl = 8192
    if L <= max_tl:
        tl, n_lt = L, 1
    else:
        n_lt = int(pl.cdiv(L, max_tl))
        tl = int(pl.cdiv(int(pl.cdiv(L, n_lt)), 128)) * 128
    L_pad = n_lt * tl

    nb_stats = 1
    for cand in (2, 1):
        if N % cand == 0 and (N // cand) >= 2:
            nb_stats = cand
            break
    n_bt_s = N // nb_stats

    nb_apply = 2 if N % 2 == 0 else 1
    n_bt_a = N // nb_apply

    if L_pad != L:
        x_p = jnp.pad(x_ncl, ((0, 0), (0, 0), (0, L_pad - L)))
    else:
        x_p = x_ncl

    grid_s = (n_bt_s, n_lt)
    n_chunks = n_bt_s * n_lt
    gparts, sxparts = pl.pallas_call(
        _stats_kernel,
        out_shape=(jax.ShapeDtypeStruct((n_chunks, C, C), jnp.float32),
                   jax.ShapeDtypeStruct((n_chunks, C, 1), jnp.float32)),
        grid=grid_s,
        in_specs=[pl.BlockSpec((nb_stats, C, tl), lambda n, l: (n, 0, l))],
        out_specs=(pl.BlockSpec((1, C, C), lambda n, l, _g=n_lt: (n * _g + l, 0, 0)),
                   pl.BlockSpec((1, C, 1), lambda n, l, _g=n_lt: (n * _g + l, 0, 0))),
        compiler_params=pltpu.CompilerParams(
            dimension_semantics=("parallel", "parallel"),
            vmem_limit_bytes=100 * 1024 * 1024),
    )(x_p)

    import functools
    w1s, shift, w2b = pl.pallas_call(
        functools.partial(_fold_kernel, 1.0 / float(N * L)),
        out_shape=(jax.ShapeDtypeStruct((H, C), jnp.bfloat16),
                   jax.ShapeDtypeStruct((H, 1), jnp.float32),
                   jax.ShapeDtypeStruct((C_out, H), jnp.bfloat16)),
        compiler_params=pltpu.CompilerParams(
            vmem_limit_bytes=100 * 1024 * 1024),
    )(gparts, sxparts,
      w1.astype(jnp.float32),
      w2.astype(jnp.float32),
      gamma.astype(jnp.float32).reshape(H, 1),
      beta.astype(jnp.float32).reshape(H, 1))

    grid_a = (n_bt_a, n_lt)
    out_p = pl.pallas_call(
        _apply_kernel,
        out_shape=jax.ShapeDtypeStruct((N, C_out, L_pad), x_ncl.dtype),
        grid=grid_a,
        in_specs=[
            pl.BlockSpec((nb_apply, C, tl), lambda n, l: (n, 0, l)),
            pl.BlockSpec((H, C), lambda n, l: (0, 0)),
            pl.BlockSpec((C_out, H), lambda n, l: (0, 0)),
            pl.BlockSpec((H, 1), lambda n, l: (0, 0)),
        ],
        out_specs=pl.BlockSpec((nb_apply, C_out, tl), lambda n, l: (n, 0, l)),
        compiler_params=pltpu.CompilerParams(
            dimension_semantics=("parallel", "parallel"),
            vmem_limit_bytes=100 * 1024 * 1024),
    )(x_p, w1s, w2b, shift)

    if L_pad != L:
        out_p = out_p[:, :, :L]
    return out_p
